# baseline (speedup 1.0000x reference)
import functools
import jax
import jax.numpy as jnp
from jax.experimental import pallas as pl
from jax.experimental.pallas import tpu as pltpu

K_PROP = 10
ALPHA = 0.1


def _mlp_body(x_ref, w1t_ref, b1_ref, w2t_ref, b2_ref, z_ref):
    h = jnp.dot(x_ref[...], w1t_ref[...], preferred_element_type=jnp.float32)
    h = jnp.maximum(h + b1_ref[...], 0.0)
    z = jnp.dot(h, w2t_ref[...], preferred_element_type=jnp.float32)
    z_ref[...] = z + b2_ref[...]


def _mlp(x, W1, b1, W2, b2):
    n, d = x.shape
    hidden = W1.shape[0]
    ncls = W2.shape[0]
    blk = 2000
    grid = (n // blk,)
    return pl.pallas_call(
        _mlp_body,
        grid=grid,
        in_specs=[
            pl.BlockSpec((blk, d), lambda i: (i, 0)),
            pl.BlockSpec((d, hidden), lambda i: (0, 0)),
            pl.BlockSpec((1, hidden), lambda i: (0, 0)),
            pl.BlockSpec((hidden, ncls), lambda i: (0, 0)),
            pl.BlockSpec((1, ncls), lambda i: (0, 0)),
        ],
        out_specs=pl.BlockSpec((blk, ncls), lambda i: (i, 0)),
        out_shape=jax.ShapeDtypeStruct((n, ncls), jnp.float32),
    )(x, W1.T, b1[None, :], W2.T, b2[None, :])


def kernel(x, edge_index, W1, b1, W2, b2):
    n = x.shape[0]
    z = _mlp(x, W1, b1, W2, b2)

    src = edge_index[0].astype(jnp.int32)
    dst = edge_index[1].astype(jnp.int32)
    deg = jnp.ones((n,), dtype=jnp.float32).at[dst].add(1.0)
    dis = jax.lax.rsqrt(deg)
    norm = dis[src] * dis[dst]

    out = z
    for _ in range(K_PROP):
        msg = norm[:, None] * out[src]
        agg = jnp.zeros_like(z).at[dst].add(msg)
        agg = agg + dis[:, None] * dis[:, None] * out
        out = (1.0 - ALPHA) * agg + ALPHA * z

    return jax.nn.log_softmax(out, axis=1)


# speedup vs baseline: 51.3672x; 51.3672x over previous
import functools
import jax
import jax.numpy as jnp
from jax import lax
from jax.experimental import pallas as pl
from jax.experimental.pallas import tpu as pltpu
from jax.experimental.pallas import tpu_sc as plsc

K_PROP = 10
ALPHA = 0.1

N_NODES = 100000
N_EDGES = 3200000
F = 16
NPAD = 100352
NF = NPAD * F
ROWS_PER_TILE = NPAD // 16
NODES_PER_W = NPAD // 32

NW = 32
CHUNK = 1024
CHUNKS_PER_W = 98
EPW = CHUNK * CHUNKS_PER_W
EPAD = EPW * NW

_SC_MESH = plsc.VectorSubcoreMesh(core_axis_name="c", subcore_axis_name="s")
_SC_PARAMS = pltpu.CompilerParams(use_tc_tiling_on_sc=False,
                                  needs_layout_passes=False)


def _mlp_body(x_ref, w1t_ref, b1_ref, w2t_ref, b2_ref, z_ref):
    h = jnp.dot(x_ref[...], w1t_ref[...], preferred_element_type=jnp.float32)
    h = jnp.maximum(h + b1_ref[...], 0.0)
    z = jnp.dot(h, w2t_ref[...], preferred_element_type=jnp.float32)
    z_ref[...] = z + b2_ref[...]


def _mlp(x, W1, b1, W2, b2):
    n, d = x.shape
    hidden = W1.shape[0]
    ncls = W2.shape[0]
    blk = 2000
    return pl.pallas_call(
        _mlp_body,
        grid=(n // blk,),
        in_specs=[
            pl.BlockSpec((blk, d), lambda i: (i, 0)),
            pl.BlockSpec((d, hidden), lambda i: (0, 0)),
            pl.BlockSpec((1, hidden), lambda i: (0, 0)),
            pl.BlockSpec((hidden, ncls), lambda i: (0, 0)),
            pl.BlockSpec((1, ncls), lambda i: (0, 0)),
        ],
        out_specs=pl.BlockSpec((blk, ncls), lambda i: (i, 0)),
        out_shape=jax.ShapeDtypeStruct((NPAD, ncls), jnp.float32),
    )(x, W1.T, b1[None, :], W2.T, b2[None, :])


def _hist_body(dst_hbm, cnt_hbm, didx_v, ones_v, zeros_v, acc_sh):
    c = lax.axis_index("c")
    s = lax.axis_index("s")
    wid = c * 16 + s
    r0 = s * ROWS_PER_TILE

    def fill(i, carry):
        ones_v[pl.ds(i * 16, 16)] = jnp.full((16,), 1.0, jnp.float32)
        zeros_v[pl.ds(i * 16, 16)] = jnp.zeros((16,), jnp.float32)
        return carry

    lax.fori_loop(0, ROWS_PER_TILE // 16, fill, 0)
    pltpu.sync_copy(zeros_v, acc_sh.at[pl.ds(r0, ROWS_PER_TILE)])
    plsc.subcore_barrier()

    e0 = wid * EPW

    def chunk(i, carry):
        pltpu.sync_copy(dst_hbm.at[pl.ds(e0 + i * CHUNK, CHUNK)], didx_v)
        pltpu.sync_copy(ones_v.at[pl.ds(0, CHUNK)], acc_sh.at[didx_v], add=True)
        return carry

    lax.fori_loop(0, CHUNKS_PER_W, chunk, 0)
    plsc.subcore_barrier()
    pltpu.sync_copy(acc_sh.at[pl.ds(r0, ROWS_PER_TILE)], zeros_v)
    pltpu.sync_copy(zeros_v, cnt_hbm.at[pl.ds(c * NPAD + r0, ROWS_PER_TILE)])


_hist = functools.partial(
    pl.kernel,
    out_type=jax.ShapeDtypeStruct((2 * NPAD,), jnp.float32),
    mesh=_SC_MESH,
    scratch_types=[
        pltpu.VMEM((CHUNK,), jnp.int32),
        pltpu.VMEM((ROWS_PER_TILE,), jnp.float32),
        pltpu.VMEM((ROWS_PER_TILE,), jnp.float32),
        pltpu.VMEM_SHARED((NPAD,), jnp.float32),
    ],
    compiler_params=_SC_PARAMS,
)(_hist_body)


def _rsqrt16(x):
    i = plsc.bitcast(x, jnp.int32)
    i = jnp.int32(0x5F3759DF) - lax.shift_right_logical(i, 1)
    y = plsc.bitcast(i, jnp.float32)
    for _ in range(3):
        y = y * (1.5 - 0.5 * x * y * y)
    return y


def _prep_body(cnt_hbm, disf_hbm, d2f_hbm, ca_v, cb_v, dis_v, d2_v, ed_v, eq_v):
    c = lax.axis_index("c")
    s = lax.axis_index("s")
    wid = c * 16 + s
    n0 = wid * NODES_PER_W
    pltpu.sync_copy(cnt_hbm.at[pl.ds(n0, NODES_PER_W)], ca_v)
    pltpu.sync_copy(cnt_hbm.at[pl.ds(NPAD + n0, NODES_PER_W)], cb_v)

    def compute(i, carry):
        o = pl.ds(i * 16, 16)
        deg = ca_v[o] + cb_v[o] + 1.0
        dis = _rsqrt16(deg)
        dis_v[o] = dis
        d2_v[o] = (1.0 - ALPHA) * dis * dis
        return carry

    lax.fori_loop(0, NODES_PER_W // 16, compute, 0)

    def expand(j, carry):
        idx = jnp.full((16,), j, jnp.int32)
        ed_v[pl.ds(j * 16, 16)] = plsc.load_gather(dis_v, [idx])
        eq_v[pl.ds(j * 16, 16)] = plsc.load_gather(d2_v, [idx])
        return carry

    lax.fori_loop(0, NODES_PER_W, expand, 0)
    pltpu.sync_copy(ed_v, disf_hbm.at[pl.ds(n0 * 16, NODES_PER_W * 16)])
    pltpu.sync_copy(eq_v, d2f_hbm.at[pl.ds(n0 * 16, NODES_PER_W * 16)])


_prep = functools.partial(
    pl.kernel,
    out_type=[
        jax.ShapeDtypeStruct((NF,), jnp.float32),
        jax.ShapeDtypeStruct((NF,), jnp.float32),
    ],
    mesh=_SC_MESH,
    scratch_types=[
        pltpu.VMEM((NODES_PER_W,), jnp.float32),
        pltpu.VMEM((NODES_PER_W,), jnp.float32),
        pltpu.VMEM((NODES_PER_W,), jnp.float32),
        pltpu.VMEM((NODES_PER_W,), jnp.float32),
        pltpu.VMEM((NODES_PER_W * 16,), jnp.float32),
        pltpu.VMEM((NODES_PER_W * 16,), jnp.float32),
    ],
    compiler_params=_SC_PARAMS,
)(_prep_body)


PIECE = ROWS_PER_TILE // 8


def _prop_body(s_hbm, src_hbm, dst_hbm, agg_hbm, sidx_v, didx_v, rows_v,
               acc_sh):
    c = lax.axis_index("c")
    s = lax.axis_index("s")
    wid = c * 16 + s
    r0 = s * ROWS_PER_TILE

    def initp(j, carry):
        o = pl.ds(r0 + j * PIECE, PIECE)
        pltpu.sync_copy(s_hbm.at[o], rows_v.at[pl.ds(0, PIECE)])
        pltpu.sync_copy(rows_v.at[pl.ds(0, PIECE)], acc_sh.at[o])
        return carry

    lax.fori_loop(0, 8, initp, 0)
    plsc.subcore_barrier()

    e0 = wid * EPW

    def chunk(i, carry):
        base = e0 + i * CHUNK
        pltpu.sync_copy(src_hbm.at[pl.ds(base, CHUNK)], sidx_v)
        pltpu.sync_copy(dst_hbm.at[pl.ds(base, CHUNK)], didx_v)
        pltpu.sync_copy(s_hbm.at[sidx_v], rows_v)
        pltpu.sync_copy(rows_v, acc_sh.at[didx_v], add=True)
        return carry

    lax.fori_loop(0, CHUNKS_PER_W, chunk, 0)
    plsc.subcore_barrier()

    def dump(j, carry):
        o = pl.ds(r0 + j * PIECE, PIECE)
        pltpu.sync_copy(acc_sh.at[o], rows_v.at[pl.ds(0, PIECE)])
        pltpu.sync_copy(rows_v.at[pl.ds(0, PIECE)],
                        agg_hbm.at[pl.ds(c * NPAD + r0 + j * PIECE, PIECE)])
        return carry

    lax.fori_loop(0, 8, dump, 0)


_prop = functools.partial(
    pl.kernel,
    out_type=jax.ShapeDtypeStruct((2 * NPAD, F), jnp.float32),
    mesh=_SC_MESH,
    scratch_types=[
        pltpu.VMEM((CHUNK,), jnp.int32),
        pltpu.VMEM((CHUNK,), jnp.int32),
        pltpu.VMEM((CHUNK, F), jnp.float32),
        pltpu.VMEM_SHARED((NPAD, F), jnp.float32),
    ],
    compiler_params=_SC_PARAMS,
)(_prop_body)


def _s0_body(disf_ref, zf_ref, s0_ref):
    s0_ref[...] = disf_ref[...] * zf_ref[...]


def _s0mul(disf, zf):
    blk = NF // 16
    return pl.pallas_call(
        _s0_body,
        grid=(16,),
        in_specs=[
            pl.BlockSpec((blk,), lambda i: (i,)),
            pl.BlockSpec((blk,), lambda i: (i,)),
        ],
        out_specs=pl.BlockSpec((blk,), lambda i: (i,)),
        out_shape=jax.ShapeDtypeStruct((NF,), jnp.float32),
    )(disf, zf)


def _upd_body(aggA_ref, aggB_ref, s_ref, s0_ref, d2_ref, out_ref):
    g = aggA_ref[...] + aggB_ref[...] - s_ref[...]
    out_ref[...] = d2_ref[...] * g + ALPHA * s0_ref[...]


def _update(aggf, sf, s0f, d2f):
    blk = NF // 16
    return pl.pallas_call(
        _upd_body,
        grid=(16,),
        in_specs=[
            pl.BlockSpec((blk,), lambda i: (i,)),
            pl.BlockSpec((blk,), lambda i: (i + 16,)),
            pl.BlockSpec((blk,), lambda i: (i,)),
            pl.BlockSpec((blk,), lambda i: (i,)),
            pl.BlockSpec((blk,), lambda i: (i,)),
        ],
        out_specs=pl.BlockSpec((blk,), lambda i: (i,)),
        out_shape=jax.ShapeDtypeStruct((NF,), jnp.float32),
    )(aggf, aggf, sf, s0f, d2f)


def _fin_body(aggA_ref, aggB_ref, s_ref, s0_ref, dis_ref, out_ref):
    dis = dis_ref[...]
    g = aggA_ref[...] + aggB_ref[...] - s_ref[...]
    s_new = (1.0 - ALPHA) * dis * dis * g + ALPHA * s0_ref[...]
    o = s_new / dis
    m = jnp.max(o, axis=1, keepdims=True)
    e = jnp.exp(o - m)
    lse = jnp.log(jnp.sum(e, axis=1, keepdims=True))
    out_ref[...] = o - m - lse


def _final(agg2d, sf, s0f, disf):
    blk = 2000
    spec = pl.BlockSpec((blk, F), lambda i: (i, 0))
    return pl.pallas_call(
        _fin_body,
        grid=(N_NODES // blk,),
        in_specs=[spec, spec, spec, spec, spec],
        out_specs=spec,
        out_shape=jax.ShapeDtypeStruct((N_NODES, F), jnp.float32),
    )(agg2d[:NPAD], agg2d[NPAD:], sf.reshape(NPAD, F), s0f.reshape(NPAD, F),
      disf.reshape(NPAD, F))


def kernel(x, edge_index, W1, b1, W2, b2):
    src = edge_index[0].astype(jnp.int32)
    dst = edge_index[1].astype(jnp.int32)
    pad_idx = jnp.full((EPAD - N_EDGES,), N_NODES, jnp.int32)
    src_p = jnp.concatenate([src, pad_idx])
    dst_p = jnp.concatenate([dst, pad_idx])

    cnt = _hist(dst_p)
    disf, d2f = _prep(cnt)
    z2d = _mlp(x, W1, b1, W2, b2)
    s0f = _s0mul(disf, z2d.reshape(-1))

    sf = s0f
    for _ in range(K_PROP - 1):
        agg2d = _prop(sf.reshape(NPAD, F), src_p, dst_p)
        sf = _update(agg2d.reshape(-1), sf, s0f, d2f)
    agg2d = _prop(sf.reshape(NPAD, F), src_p, dst_p)
    return _final(agg2d, sf, s0f, disf)


# speedup vs baseline: 84.8512x; 1.6519x over previous
import functools
import jax
import jax.numpy as jnp
from jax import lax
from jax.experimental import pallas as pl
from jax.experimental.pallas import tpu as pltpu
from jax.experimental.pallas import tpu_sc as plsc

K_PROP = 10
ALPHA = 0.1

N_NODES = 100000
N_EDGES = 3200000
F = 16
NPAD = 100352
NF = NPAD * F
ROWS_PER_TILE = NPAD // 16
NODES_PER_W = NPAD // 32

NW = 32
CHUNK = 784
CHUNKS_PER_W = 128
GROUP = 16
EPW = CHUNK * CHUNKS_PER_W
EPAD = EPW * NW

_SC_MESH = plsc.VectorSubcoreMesh(core_axis_name="c", subcore_axis_name="s")
_SC_PARAMS = pltpu.CompilerParams(use_tc_tiling_on_sc=False,
                                  needs_layout_passes=False)


def _mlp_body(x_ref, w1t_ref, b1_ref, w2t_ref, b2_ref, z_ref):
    h = jnp.dot(x_ref[...], w1t_ref[...], preferred_element_type=jnp.float32)
    h = jnp.maximum(h + b1_ref[...], 0.0)
    z = jnp.dot(h, w2t_ref[...], preferred_element_type=jnp.float32)
    z_ref[...] = z + b2_ref[...]


def _mlp(x, W1, b1, W2, b2):
    n, d = x.shape
    hidden = W1.shape[0]
    ncls = W2.shape[0]
    blk = 2000
    return pl.pallas_call(
        _mlp_body,
        grid=(n // blk,),
        in_specs=[
            pl.BlockSpec((blk, d), lambda i: (i, 0)),
            pl.BlockSpec((d, hidden), lambda i: (0, 0)),
            pl.BlockSpec((1, hidden), lambda i: (0, 0)),
            pl.BlockSpec((hidden, ncls), lambda i: (0, 0)),
            pl.BlockSpec((1, ncls), lambda i: (0, 0)),
        ],
        out_specs=pl.BlockSpec((blk, ncls), lambda i: (i, 0)),
        out_shape=jax.ShapeDtypeStruct((NPAD, ncls), jnp.float32),
    )(x, W1.T, b1[None, :], W2.T, b2[None, :])


def _hist_body(dst_hbm, cnt_hbm, didx_v, ones_v, zeros_v, acc_sh):
    c = lax.axis_index("c")
    s = lax.axis_index("s")
    wid = c * 16 + s
    r0 = s * ROWS_PER_TILE

    def fill(i, carry):
        ones_v[pl.ds(i * 16, 16)] = jnp.full((16,), 1.0, jnp.float32)
        zeros_v[pl.ds(i * 16, 16)] = jnp.zeros((16,), jnp.float32)
        return carry

    lax.fori_loop(0, ROWS_PER_TILE // 16, fill, 0)
    pltpu.sync_copy(zeros_v, acc_sh.at[pl.ds(r0, ROWS_PER_TILE)])
    plsc.subcore_barrier()

    e0 = wid * EPW

    def chunk(i, carry):
        pltpu.sync_copy(dst_hbm.at[pl.ds(e0 + i * CHUNK, CHUNK)], didx_v)
        pltpu.sync_copy(ones_v.at[pl.ds(0, CHUNK)], acc_sh.at[didx_v], add=True)
        return carry

    lax.fori_loop(0, CHUNKS_PER_W, chunk, 0)
    plsc.subcore_barrier()
    pltpu.sync_copy(acc_sh.at[pl.ds(r0, ROWS_PER_TILE)], zeros_v)
    pltpu.sync_copy(zeros_v, cnt_hbm.at[pl.ds(c * NPAD + r0, ROWS_PER_TILE)])


_hist = functools.partial(
    pl.kernel,
    out_type=jax.ShapeDtypeStruct((2 * NPAD,), jnp.float32),
    mesh=_SC_MESH,
    scratch_types=[
        pltpu.VMEM((CHUNK,), jnp.int32),
        pltpu.VMEM((ROWS_PER_TILE,), jnp.float32),
        pltpu.VMEM((ROWS_PER_TILE,), jnp.float32),
        pltpu.VMEM_SHARED((NPAD,), jnp.float32),
    ],
    compiler_params=_SC_PARAMS,
)(_hist_body)


def _rsqrt16(x):
    i = plsc.bitcast(x, jnp.int32)
    i = jnp.int32(0x5F3759DF) - lax.shift_right_logical(i, 1)
    y = plsc.bitcast(i, jnp.float32)
    for _ in range(3):
        y = y * (1.5 - 0.5 * x * y * y)
    return y


def _prep_body(cnt_hbm, disf_hbm, d2f_hbm, ca_v, cb_v, dis_v, d2_v, ed_v, eq_v):
    c = lax.axis_index("c")
    s = lax.axis_index("s")
    wid = c * 16 + s
    n0 = wid * NODES_PER_W
    pltpu.sync_copy(cnt_hbm.at[pl.ds(n0, NODES_PER_W)], ca_v)
    pltpu.sync_copy(cnt_hbm.at[pl.ds(NPAD + n0, NODES_PER_W)], cb_v)

    def compute(i, carry):
        o = pl.ds(i * 16, 16)
        deg = ca_v[o] + cb_v[o] + 1.0
        dis = _rsqrt16(deg)
        dis_v[o] = dis
        d2_v[o] = (1.0 - ALPHA) * dis * dis
        return carry

    lax.fori_loop(0, NODES_PER_W // 16, compute, 0)

    def expand(j, carry):
        idx = jnp.full((16,), j, jnp.int32)
        ed_v[pl.ds(j * 16, 16)] = plsc.load_gather(dis_v, [idx])
        eq_v[pl.ds(j * 16, 16)] = plsc.load_gather(d2_v, [idx])
        return carry

    lax.fori_loop(0, NODES_PER_W, expand, 0)
    pltpu.sync_copy(ed_v, disf_hbm.at[pl.ds(n0 * 16, NODES_PER_W * 16)])
    pltpu.sync_copy(eq_v, d2f_hbm.at[pl.ds(n0 * 16, NODES_PER_W * 16)])


_prep = functools.partial(
    pl.kernel,
    out_type=[
        jax.ShapeDtypeStruct((NF,), jnp.float32),
        jax.ShapeDtypeStruct((NF,), jnp.float32),
    ],
    mesh=_SC_MESH,
    scratch_types=[
        pltpu.VMEM((NODES_PER_W,), jnp.float32),
        pltpu.VMEM((NODES_PER_W,), jnp.float32),
        pltpu.VMEM((NODES_PER_W,), jnp.float32),
        pltpu.VMEM((NODES_PER_W,), jnp.float32),
        pltpu.VMEM((NODES_PER_W * 16,), jnp.float32),
        pltpu.VMEM((NODES_PER_W * 16,), jnp.float32),
    ],
    compiler_params=_SC_PARAMS,
)(_prep_body)


PIECE = ROWS_PER_TILE // 8


def _prop_body(s_hbm, src_hbm, dst_hbm, agg_hbm,
               sidx0, sidx1, sidx2, didx0, didx1, didx2, rows0, rows1,
               isem0, isem1, isem2, gsem0, gsem1, ssem0, ssem1, acc_sh):
    c = lax.axis_index("c")
    s = lax.axis_index("s")
    wid = c * 16 + s
    r0 = s * ROWS_PER_TILE
    sidx = (sidx0, sidx1, sidx2)
    didx = (didx0, didx1, didx2)
    rows = (rows0, rows1)
    isem = (isem0, isem1, isem2)
    gsem = (gsem0, gsem1)
    ssem = (ssem0, ssem1)

    def fillz(j, carry):
        rows0[j, :] = jnp.zeros((16,), jnp.float32)
        return carry

    lax.fori_loop(0, CHUNK, fillz, 0)
    for j in range(8):
        pltpu.sync_copy(rows0, acc_sh.at[pl.ds(r0 + j * PIECE, PIECE)])
    plsc.subcore_barrier()

    e0 = wid * EPW

    def idx_issue(base, t):
        off = base + t * CHUNK
        k = t % 3
        return (pltpu.async_copy(src_hbm.at[pl.ds(off, CHUNK)], sidx[k],
                                 isem[k]),
                pltpu.async_copy(dst_hbm.at[pl.ds(off, CHUNK)], didx[k],
                                 isem[k]))

    def group(g, carry):
        base = e0 + g * GROUP * CHUNK
        hidx = [None] * (GROUP + 1)
        hg = [None] * GROUP
        hs = [None] * GROUP
        hidx[0] = idx_issue(base, 0)
        for t in range(GROUP):
            b = t % 2
            if t >= 2:
                hs[t - 2].wait()
            for h in hidx[t]:
                h.wait()
            hg[t] = pltpu.async_copy(s_hbm.at[sidx[t % 3]], rows[b], gsem[b])
            if t + 1 < GROUP:
                hidx[t + 1] = idx_issue(base, t + 1)
            if t >= 1:
                hg[t - 1].wait()
                hs[t - 1] = pltpu.async_copy(
                    rows[(t - 1) % 2], acc_sh.at[didx[(t - 1) % 3]],
                    ssem[(t - 1) % 2], add=True)
        hg[GROUP - 1].wait()
        hs[GROUP - 1] = pltpu.async_copy(
            rows[(GROUP - 1) % 2], acc_sh.at[didx[(GROUP - 1) % 3]],
            ssem[(GROUP - 1) % 2], add=True)
        hs[GROUP - 2].wait()
        hs[GROUP - 1].wait()
        return carry

    lax.fori_loop(0, CHUNKS_PER_W // GROUP, group, 0)
    plsc.subcore_barrier()

    ha = [None] * 8
    hw = [None] * 8
    for j in range(8):
        b = j % 2
        if j >= 2:
            hw[j - 2].wait()
        ha[j] = pltpu.async_copy(acc_sh.at[pl.ds(r0 + j * PIECE, PIECE)],
                                 rows[b], gsem[b])
        if j >= 1:
            ha[j - 1].wait()
            hw[j - 1] = pltpu.async_copy(
                rows[(j - 1) % 2],
                agg_hbm.at[pl.ds(c * NPAD + r0 + (j - 1) * PIECE, PIECE)],
                ssem[(j - 1) % 2])
    ha[7].wait()
    hw[7] = pltpu.async_copy(rows[1],
                             agg_hbm.at[pl.ds(c * NPAD + r0 + 7 * PIECE,
                                              PIECE)], ssem[1])
    hw[6].wait()
    hw[7].wait()


_prop = functools.partial(
    pl.kernel,
    out_type=jax.ShapeDtypeStruct((2 * NPAD, F), jnp.float32),
    mesh=_SC_MESH,
    scratch_types=[
        pltpu.VMEM((CHUNK,), jnp.int32),
        pltpu.VMEM((CHUNK,), jnp.int32),
        pltpu.VMEM((CHUNK,), jnp.int32),
        pltpu.VMEM((CHUNK,), jnp.int32),
        pltpu.VMEM((CHUNK,), jnp.int32),
        pltpu.VMEM((CHUNK,), jnp.int32),
        pltpu.VMEM((CHUNK, F), jnp.float32),
        pltpu.VMEM((CHUNK, F), jnp.float32),
        pltpu.SemaphoreType.DMA,
        pltpu.SemaphoreType.DMA,
        pltpu.SemaphoreType.DMA,
        pltpu.SemaphoreType.DMA,
        pltpu.SemaphoreType.DMA,
        pltpu.SemaphoreType.DMA,
        pltpu.SemaphoreType.DMA,
        pltpu.VMEM_SHARED((NPAD, F), jnp.float32),
    ],
    compiler_params=_SC_PARAMS,
)(_prop_body)


def _s0_body(disf_ref, zf_ref, s0_ref):
    s0_ref[...] = disf_ref[...] * zf_ref[...]


def _s0mul(disf, zf):
    blk = NF // 16
    return pl.pallas_call(
        _s0_body,
        grid=(16,),
        in_specs=[
            pl.BlockSpec((blk,), lambda i: (i,)),
            pl.BlockSpec((blk,), lambda i: (i,)),
        ],
        out_specs=pl.BlockSpec((blk,), lambda i: (i,)),
        out_shape=jax.ShapeDtypeStruct((NF,), jnp.float32),
    )(disf, zf)


def _upd_body(aggA_ref, aggB_ref, s_ref, s0_ref, d2_ref, out_ref):
    g = aggA_ref[...] + aggB_ref[...] + s_ref[...]
    out_ref[...] = d2_ref[...] * g + ALPHA * s0_ref[...]


def _update(aggf, sf, s0f, d2f):
    blk = NF // 16
    return pl.pallas_call(
        _upd_body,
        grid=(16,),
        in_specs=[
            pl.BlockSpec((blk,), lambda i: (i,)),
            pl.BlockSpec((blk,), lambda i: (i + 16,)),
            pl.BlockSpec((blk,), lambda i: (i,)),
            pl.BlockSpec((blk,), lambda i: (i,)),
            pl.BlockSpec((blk,), lambda i: (i,)),
        ],
        out_specs=pl.BlockSpec((blk,), lambda i: (i,)),
        out_shape=jax.ShapeDtypeStruct((NF,), jnp.float32),
    )(aggf, aggf, sf, s0f, d2f)


def _fin_body(aggA_ref, aggB_ref, s_ref, s0_ref, dis_ref, out_ref):
    dis = dis_ref[...]
    g = aggA_ref[...] + aggB_ref[...] + s_ref[...]
    s_new = (1.0 - ALPHA) * dis * dis * g + ALPHA * s0_ref[...]
    o = s_new / dis
    m = jnp.max(o, axis=1, keepdims=True)
    e = jnp.exp(o - m)
    lse = jnp.log(jnp.sum(e, axis=1, keepdims=True))
    out_ref[...] = o - m - lse


def _final(agg2d, sf, s0f, disf):
    blk = 2000
    spec = pl.BlockSpec((blk, F), lambda i: (i, 0))
    return pl.pallas_call(
        _fin_body,
        grid=(N_NODES // blk,),
        in_specs=[spec, spec, spec, spec, spec],
        out_specs=spec,
        out_shape=jax.ShapeDtypeStruct((N_NODES, F), jnp.float32),
    )(agg2d[:NPAD], agg2d[NPAD:], sf.reshape(NPAD, F), s0f.reshape(NPAD, F),
      disf.reshape(NPAD, F))


def kernel(x, edge_index, W1, b1, W2, b2):
    src = edge_index[0].astype(jnp.int32)
    dst = edge_index[1].astype(jnp.int32)
    pad_idx = jnp.full((EPAD - N_EDGES,), N_NODES, jnp.int32)
    src_p = jnp.concatenate([src, pad_idx])
    dst_p = jnp.concatenate([dst, pad_idx])

    cnt = _hist(dst_p)
    disf, d2f = _prep(cnt)
    z2d = _mlp(x, W1, b1, W2, b2)
    s0f = _s0mul(disf, z2d.reshape(-1))

    sf = s0f
    for _ in range(K_PROP - 1):
        agg2d = _prop(sf.reshape(NPAD, F), src_p, dst_p)
        sf = _update(agg2d.reshape(-1), sf, s0f, d2f)
    agg2d = _prop(sf.reshape(NPAD, F), src_p, dst_p)
    return _final(agg2d, sf, s0f, disf)


# speedup vs baseline: 85.8952x; 1.0123x over previous
import functools
import jax
import jax.numpy as jnp
from jax import lax
from jax.experimental import pallas as pl
from jax.experimental.pallas import tpu as pltpu
from jax.experimental.pallas import tpu_sc as plsc

K_PROP = 10
ALPHA = 0.1

N_NODES = 100000
N_EDGES = 3200000
F = 16
NPAD = 100352
NF = NPAD * F
ROWS_PER_TILE = NPAD // 16
NODES_PER_W = NPAD // 32

NW = 32
CHUNK = 784
CHUNKS_PER_W = 128
GROUP = 32
EPW = CHUNK * CHUNKS_PER_W
EPAD = EPW * NW

_SC_MESH = plsc.VectorSubcoreMesh(core_axis_name="c", subcore_axis_name="s")
_SC_PARAMS = pltpu.CompilerParams(use_tc_tiling_on_sc=False,
                                  needs_layout_passes=False)


def _mlp_body(x_ref, w1t_ref, b1_ref, w2t_ref, b2_ref, z_ref):
    h = jnp.dot(x_ref[...], w1t_ref[...], preferred_element_type=jnp.float32)
    h = jnp.maximum(h + b1_ref[...], 0.0)
    z = jnp.dot(h, w2t_ref[...], preferred_element_type=jnp.float32)
    z_ref[...] = z + b2_ref[...]


def _mlp(x, W1, b1, W2, b2):
    n, d = x.shape
    hidden = W1.shape[0]
    ncls = W2.shape[0]
    blk = 2000
    return pl.pallas_call(
        _mlp_body,
        grid=(n // blk,),
        in_specs=[
            pl.BlockSpec((blk, d), lambda i: (i, 0)),
            pl.BlockSpec((d, hidden), lambda i: (0, 0)),
            pl.BlockSpec((1, hidden), lambda i: (0, 0)),
            pl.BlockSpec((hidden, ncls), lambda i: (0, 0)),
            pl.BlockSpec((1, ncls), lambda i: (0, 0)),
        ],
        out_specs=pl.BlockSpec((blk, ncls), lambda i: (i, 0)),
        out_shape=jax.ShapeDtypeStruct((NPAD, ncls), jnp.float32),
    )(x, W1.T, b1[None, :], W2.T, b2[None, :])


def _hist_body(dst_hbm, cnt_hbm, didx_v, ones_v, zeros_v, acc_sh):
    c = lax.axis_index("c")
    s = lax.axis_index("s")
    wid = c * 16 + s
    r0 = s * ROWS_PER_TILE

    def fill(i, carry):
        ones_v[pl.ds(i * 16, 16)] = jnp.full((16,), 1.0, jnp.float32)
        zeros_v[pl.ds(i * 16, 16)] = jnp.zeros((16,), jnp.float32)
        return carry

    lax.fori_loop(0, ROWS_PER_TILE // 16, fill, 0)
    pltpu.sync_copy(zeros_v, acc_sh.at[pl.ds(r0, ROWS_PER_TILE)])
    plsc.subcore_barrier()

    e0 = wid * EPW

    def chunk(i, carry):
        pltpu.sync_copy(dst_hbm.at[pl.ds(e0 + i * CHUNK, CHUNK)], didx_v)
        pltpu.sync_copy(ones_v.at[pl.ds(0, CHUNK)], acc_sh.at[didx_v], add=True)
        return carry

    lax.fori_loop(0, CHUNKS_PER_W, chunk, 0)
    plsc.subcore_barrier()
    pltpu.sync_copy(acc_sh.at[pl.ds(r0, ROWS_PER_TILE)], zeros_v)
    pltpu.sync_copy(zeros_v, cnt_hbm.at[pl.ds(c * NPAD + r0, ROWS_PER_TILE)])


_hist = functools.partial(
    pl.kernel,
    out_type=jax.ShapeDtypeStruct((2 * NPAD,), jnp.float32),
    mesh=_SC_MESH,
    scratch_types=[
        pltpu.VMEM((CHUNK,), jnp.int32),
        pltpu.VMEM((ROWS_PER_TILE,), jnp.float32),
        pltpu.VMEM((ROWS_PER_TILE,), jnp.float32),
        pltpu.VMEM_SHARED((NPAD,), jnp.float32),
    ],
    compiler_params=_SC_PARAMS,
)(_hist_body)


def _rsqrt16(x):
    i = plsc.bitcast(x, jnp.int32)
    i = jnp.int32(0x5F3759DF) - lax.shift_right_logical(i, 1)
    y = plsc.bitcast(i, jnp.float32)
    for _ in range(3):
        y = y * (1.5 - 0.5 * x * y * y)
    return y


def _prep_body(cnt_hbm, disf_hbm, d2f_hbm, ca_v, cb_v, dis_v, d2_v, ed_v, eq_v):
    c = lax.axis_index("c")
    s = lax.axis_index("s")
    wid = c * 16 + s
    n0 = wid * NODES_PER_W
    pltpu.sync_copy(cnt_hbm.at[pl.ds(n0, NODES_PER_W)], ca_v)
    pltpu.sync_copy(cnt_hbm.at[pl.ds(NPAD + n0, NODES_PER_W)], cb_v)

    def compute(i, carry):
        o = pl.ds(i * 16, 16)
        deg = ca_v[o] + cb_v[o] + 1.0
        dis = _rsqrt16(deg)
        dis_v[o] = dis
        d2_v[o] = (1.0 - ALPHA) * dis * dis
        return carry

    lax.fori_loop(0, NODES_PER_W // 16, compute, 0)

    def expand(j, carry):
        idx = jnp.full((16,), j, jnp.int32)
        ed_v[pl.ds(j * 16, 16)] = plsc.load_gather(dis_v, [idx])
        eq_v[pl.ds(j * 16, 16)] = plsc.load_gather(d2_v, [idx])
        return carry

    lax.fori_loop(0, NODES_PER_W, expand, 0)
    pltpu.sync_copy(ed_v, disf_hbm.at[pl.ds(n0 * 16, NODES_PER_W * 16)])
    pltpu.sync_copy(eq_v, d2f_hbm.at[pl.ds(n0 * 16, NODES_PER_W * 16)])


_prep = functools.partial(
    pl.kernel,
    out_type=[
        jax.ShapeDtypeStruct((NF,), jnp.float32),
        jax.ShapeDtypeStruct((NF,), jnp.float32),
    ],
    mesh=_SC_MESH,
    scratch_types=[
        pltpu.VMEM((NODES_PER_W,), jnp.float32),
        pltpu.VMEM((NODES_PER_W,), jnp.float32),
        pltpu.VMEM((NODES_PER_W,), jnp.float32),
        pltpu.VMEM((NODES_PER_W,), jnp.float32),
        pltpu.VMEM((NODES_PER_W * 16,), jnp.float32),
        pltpu.VMEM((NODES_PER_W * 16,), jnp.float32),
    ],
    compiler_params=_SC_PARAMS,
)(_prep_body)


PIECE = ROWS_PER_TILE // 8


def _prop_body(s_hbm, src_hbm, dst_hbm, agg_hbm,
               sidx0, sidx1, sidx2, didx0, didx1, didx2, rows0, rows1,
               isem0, isem1, isem2, gsem0, gsem1, ssem0, ssem1, acc_sh):
    c = lax.axis_index("c")
    s = lax.axis_index("s")
    wid = c * 16 + s
    r0 = s * ROWS_PER_TILE
    sidx = (sidx0, sidx1, sidx2)
    didx = (didx0, didx1, didx2)
    rows = (rows0, rows1)
    isem = (isem0, isem1, isem2)
    gsem = (gsem0, gsem1)
    ssem = (ssem0, ssem1)

    def fillz(j, carry):
        rows0[j, :] = jnp.zeros((16,), jnp.float32)
        return carry

    lax.fori_loop(0, CHUNK, fillz, 0)
    for j in range(8):
        pltpu.sync_copy(rows0, acc_sh.at[pl.ds(r0 + j * PIECE, PIECE)])
    plsc.subcore_barrier()

    e0 = wid * EPW

    def idx_issue(base, t):
        off = base + t * CHUNK
        k = t % 3
        return (pltpu.async_copy(src_hbm.at[pl.ds(off, CHUNK)], sidx[k],
                                 isem[k]),
                pltpu.async_copy(dst_hbm.at[pl.ds(off, CHUNK)], didx[k],
                                 isem[k]))

    def group(g, carry):
        base = e0 + g * GROUP * CHUNK
        hidx = [None] * (GROUP + 1)
        hg = [None] * GROUP
        hs = [None] * GROUP
        hidx[0] = idx_issue(base, 0)
        for t in range(GROUP):
            b = t % 2
            if t >= 2:
                hs[t - 2].wait()
            for h in hidx[t]:
                h.wait()
            hg[t] = pltpu.async_copy(s_hbm.at[sidx[t % 3]], rows[b], gsem[b])
            if t + 1 < GROUP:
                hidx[t + 1] = idx_issue(base, t + 1)
            if t >= 1:
                hg[t - 1].wait()
                hs[t - 1] = pltpu.async_copy(
                    rows[(t - 1) % 2], acc_sh.at[didx[(t - 1) % 3]],
                    ssem[(t - 1) % 2], add=True)
        hg[GROUP - 1].wait()
        hs[GROUP - 1] = pltpu.async_copy(
            rows[(GROUP - 1) % 2], acc_sh.at[didx[(GROUP - 1) % 3]],
            ssem[(GROUP - 1) % 2], add=True)
        hs[GROUP - 2].wait()
        hs[GROUP - 1].wait()
        return carry

    lax.fori_loop(0, CHUNKS_PER_W // GROUP, group, 0)
    plsc.subcore_barrier()

    ha = [None] * 8
    hw = [None] * 8
    for j in range(8):
        b = j % 2
        if j >= 2:
            hw[j - 2].wait()
        ha[j] = pltpu.async_copy(acc_sh.at[pl.ds(r0 + j * PIECE, PIECE)],
                                 rows[b], gsem[b])
        if j >= 1:
            ha[j - 1].wait()
            hw[j - 1] = pltpu.async_copy(
                rows[(j - 1) % 2],
                agg_hbm.at[pl.ds(c * NPAD + r0 + (j - 1) * PIECE, PIECE)],
                ssem[(j - 1) % 2])
    ha[7].wait()
    hw[7] = pltpu.async_copy(rows[1],
                             agg_hbm.at[pl.ds(c * NPAD + r0 + 7 * PIECE,
                                              PIECE)], ssem[1])
    hw[6].wait()
    hw[7].wait()


_prop = functools.partial(
    pl.kernel,
    out_type=jax.ShapeDtypeStruct((2 * NPAD, F), jnp.float32),
    mesh=_SC_MESH,
    scratch_types=[
        pltpu.VMEM((CHUNK,), jnp.int32),
        pltpu.VMEM((CHUNK,), jnp.int32),
        pltpu.VMEM((CHUNK,), jnp.int32),
        pltpu.VMEM((CHUNK,), jnp.int32),
        pltpu.VMEM((CHUNK,), jnp.int32),
        pltpu.VMEM((CHUNK,), jnp.int32),
        pltpu.VMEM((CHUNK, F), jnp.float32),
        pltpu.VMEM((CHUNK, F), jnp.float32),
        pltpu.SemaphoreType.DMA,
        pltpu.SemaphoreType.DMA,
        pltpu.SemaphoreType.DMA,
        pltpu.SemaphoreType.DMA,
        pltpu.SemaphoreType.DMA,
        pltpu.SemaphoreType.DMA,
        pltpu.SemaphoreType.DMA,
        pltpu.VMEM_SHARED((NPAD, F), jnp.float32),
    ],
    compiler_params=_SC_PARAMS,
)(_prop_body)


def _s0_body(disf_ref, zf_ref, s0_ref):
    s0_ref[...] = disf_ref[...] * zf_ref[...]


def _s0mul(disf, zf):
    blk = NF // 16
    return pl.pallas_call(
        _s0_body,
        grid=(16,),
        in_specs=[
            pl.BlockSpec((blk,), lambda i: (i,)),
            pl.BlockSpec((blk,), lambda i: (i,)),
        ],
        out_specs=pl.BlockSpec((blk,), lambda i: (i,)),
        out_shape=jax.ShapeDtypeStruct((NF,), jnp.float32),
    )(disf, zf)


def _upd_body(aggA_ref, aggB_ref, s_ref, s0_ref, d2_ref, out_ref):
    g = aggA_ref[...] + aggB_ref[...] + s_ref[...]
    out_ref[...] = d2_ref[...] * g + ALPHA * s0_ref[...]


def _update(aggf, sf, s0f, d2f):
    blk = NF // 16
    return pl.pallas_call(
        _upd_body,
        grid=(16,),
        in_specs=[
            pl.BlockSpec((blk,), lambda i: (i,)),
            pl.BlockSpec((blk,), lambda i: (i + 16,)),
            pl.BlockSpec((blk,), lambda i: (i,)),
            pl.BlockSpec((blk,), lambda i: (i,)),
            pl.BlockSpec((blk,), lambda i: (i,)),
        ],
        out_specs=pl.BlockSpec((blk,), lambda i: (i,)),
        out_shape=jax.ShapeDtypeStruct((NF,), jnp.float32),
    )(aggf, aggf, sf, s0f, d2f)


def _fin_body(aggA_ref, aggB_ref, s_ref, s0_ref, dis_ref, out_ref):
    dis = dis_ref[...]
    g = aggA_ref[...] + aggB_ref[...] + s_ref[...]
    s_new = (1.0 - ALPHA) * dis * dis * g + ALPHA * s0_ref[...]
    o = s_new / dis
    m = jnp.max(o, axis=1, keepdims=True)
    e = jnp.exp(o - m)
    lse = jnp.log(jnp.sum(e, axis=1, keepdims=True))
    out_ref[...] = o - m - lse


def _final(agg2d, sf, s0f, disf):
    blk = 2000
    spec = pl.BlockSpec((blk, F), lambda i: (i, 0))
    return pl.pallas_call(
        _fin_body,
        grid=(N_NODES // blk,),
        in_specs=[spec, spec, spec, spec, spec],
        out_specs=spec,
        out_shape=jax.ShapeDtypeStruct((N_NODES, F), jnp.float32),
    )(agg2d[:NPAD], agg2d[NPAD:], sf.reshape(NPAD, F), s0f.reshape(NPAD, F),
      disf.reshape(NPAD, F))


def kernel(x, edge_index, W1, b1, W2, b2):
    src = edge_index[0].astype(jnp.int32)
    dst = edge_index[1].astype(jnp.int32)
    pad_idx = jnp.full((EPAD - N_EDGES,), N_NODES, jnp.int32)
    src_p = jnp.concatenate([src, pad_idx])
    dst_p = jnp.concatenate([dst, pad_idx])

    cnt = _hist(dst_p)
    disf, d2f = _prep(cnt)
    z2d = _mlp(x, W1, b1, W2, b2)
    s0f = _s0mul(disf, z2d.reshape(-1))

    sf = s0f
    for _ in range(K_PROP - 1):
        agg2d = _prop(sf.reshape(NPAD, F), src_p, dst_p)
        sf = _update(agg2d.reshape(-1), sf, s0f, d2f)
    agg2d = _prop(sf.reshape(NPAD, F), src_p, dst_p)
    return _final(agg2d, sf, s0f, disf)


# speedup vs baseline: 90.0284x; 1.0481x over previous
import functools
import jax
import jax.numpy as jnp
from jax import lax
from jax.experimental import pallas as pl
from jax.experimental.pallas import tpu as pltpu
from jax.experimental.pallas import tpu_sc as plsc

K_PROP = 10
ALPHA = 0.1

N_NODES = 100000
N_EDGES = 3200000
F = 16
NPAD = 100352
NF = NPAD * F
ROWS_PER_TILE = NPAD // 16
NODES_PER_W = NPAD // 32

NW = 32
CHUNK = 784
CHUNKS_PER_W = 128
GROUP = 32
EPW = CHUNK * CHUNKS_PER_W
EPAD = EPW * NW

_SC_MESH = plsc.VectorSubcoreMesh(core_axis_name="c", subcore_axis_name="s")
_SC_PARAMS = pltpu.CompilerParams(use_tc_tiling_on_sc=False,
                                  needs_layout_passes=False)


L2 = NF // 128


def _mlp_body(x_ref, w1t_ref, b1_ref, w2t_ref, b2_ref, z_ref):
    h = jnp.dot(x_ref[...], w1t_ref[...], preferred_element_type=jnp.float32)
    h = jnp.maximum(h + b1_ref[...], 0.0)
    z = jnp.dot(h, w2t_ref[...], preferred_element_type=jnp.float32)
    z_ref[...] = z + b2_ref[...]


def _mlp(x, W1, b1, W2, b2):
    n, d = x.shape
    hidden = W1.shape[0]
    ncls = W2.shape[0]
    blk = 2000
    return pl.pallas_call(
        _mlp_body,
        grid=(n // blk,),
        in_specs=[
            pl.BlockSpec((blk, d), lambda i: (i, 0)),
            pl.BlockSpec((d, hidden), lambda i: (0, 0)),
            pl.BlockSpec((1, hidden), lambda i: (0, 0)),
            pl.BlockSpec((hidden, ncls), lambda i: (0, 0)),
            pl.BlockSpec((1, ncls), lambda i: (0, 0)),
        ],
        out_specs=pl.BlockSpec((blk, ncls), lambda i: (i, 0)),
        out_shape=jax.ShapeDtypeStruct((NPAD, ncls), jnp.float32),
    )(x, W1.T, b1[None, :], W2.T, b2[None, :])


def _s0_body(disf_ref, zf_ref, s0_ref):
    s0_ref[...] = disf_ref[...] * zf_ref[...]


def _s0mul(disf2, zf2):
    blk = L2 // 49
    spec = pl.BlockSpec((blk, 128), lambda i: (i, 0))
    return pl.pallas_call(
        _s0_body,
        grid=(49,),
        in_specs=[spec, spec],
        out_specs=spec,
        out_shape=jax.ShapeDtypeStruct((L2, 128), jnp.float32),
    )(disf2, zf2)


def _hist_body(dst_hbm, cnt_hbm, didx_v, ones_v, zeros_v, acc_sh):
    c = lax.axis_index("c")
    s = lax.axis_index("s")
    wid = c * 16 + s
    r0 = s * ROWS_PER_TILE

    def fill(i, carry):
        ones_v[pl.ds(i * 16, 16)] = jnp.full((16,), 1.0, jnp.float32)
        zeros_v[pl.ds(i * 16, 16)] = jnp.zeros((16,), jnp.float32)
        return carry

    lax.fori_loop(0, ROWS_PER_TILE // 16, fill, 0)
    pltpu.sync_copy(zeros_v, acc_sh.at[pl.ds(r0, ROWS_PER_TILE)])
    plsc.subcore_barrier()

    e0 = wid * EPW

    def chunk(i, carry):
        pltpu.sync_copy(dst_hbm.at[pl.ds(e0 + i * CHUNK, CHUNK)], didx_v)
        pltpu.sync_copy(ones_v.at[pl.ds(0, CHUNK)], acc_sh.at[didx_v], add=True)
        return carry

    lax.fori_loop(0, CHUNKS_PER_W, chunk, 0)
    plsc.subcore_barrier()
    pltpu.sync_copy(acc_sh.at[pl.ds(r0, ROWS_PER_TILE)], zeros_v)
    pltpu.sync_copy(zeros_v, cnt_hbm.at[pl.ds(c * NPAD + r0, ROWS_PER_TILE)])


_hist = functools.partial(
    pl.kernel,
    out_type=jax.ShapeDtypeStruct((2 * NPAD,), jnp.float32),
    mesh=_SC_MESH,
    scratch_types=[
        pltpu.VMEM((CHUNK,), jnp.int32),
        pltpu.VMEM((ROWS_PER_TILE,), jnp.float32),
        pltpu.VMEM((ROWS_PER_TILE,), jnp.float32),
        pltpu.VMEM_SHARED((NPAD,), jnp.float32),
    ],
    compiler_params=_SC_PARAMS,
)(_hist_body)


def _rsqrt16(x):
    i = plsc.bitcast(x, jnp.int32)
    i = jnp.int32(0x5F3759DF) - lax.shift_right_logical(i, 1)
    y = plsc.bitcast(i, jnp.float32)
    for _ in range(3):
        y = y * (1.5 - 0.5 * x * y * y)
    return y


def _prep_body(cnt_hbm, disf_hbm, d2f_hbm, ca_v, cb_v, dis_v, d2_v, ed_v, eq_v):
    c = lax.axis_index("c")
    s = lax.axis_index("s")
    wid = c * 16 + s
    n0 = wid * NODES_PER_W
    pltpu.sync_copy(cnt_hbm.at[pl.ds(n0, NODES_PER_W)], ca_v)
    pltpu.sync_copy(cnt_hbm.at[pl.ds(NPAD + n0, NODES_PER_W)], cb_v)

    def compute(i, carry):
        o = pl.ds(i * 16, 16)
        deg = ca_v[o] + cb_v[o] + 1.0
        dis = _rsqrt16(deg)
        dis_v[o] = dis
        d2_v[o] = (1.0 - ALPHA) * dis * dis
        return carry

    lax.fori_loop(0, NODES_PER_W // 16, compute, 0)

    def expand(j, carry):
        idx = jnp.full((16,), j, jnp.int32)
        ed_v[pl.ds(j * 16, 16)] = plsc.load_gather(dis_v, [idx])
        eq_v[pl.ds(j * 16, 16)] = plsc.load_gather(d2_v, [idx])
        return carry

    lax.fori_loop(0, NODES_PER_W, expand, 0)
    pltpu.sync_copy(ed_v, disf_hbm.at[pl.ds(n0 * 16, NODES_PER_W * 16)])
    pltpu.sync_copy(eq_v, d2f_hbm.at[pl.ds(n0 * 16, NODES_PER_W * 16)])


_prep = functools.partial(
    pl.kernel,
    out_type=[
        jax.ShapeDtypeStruct((NF,), jnp.float32),
        jax.ShapeDtypeStruct((NF,), jnp.float32),
    ],
    mesh=_SC_MESH,
    scratch_types=[
        pltpu.VMEM((NODES_PER_W,), jnp.float32),
        pltpu.VMEM((NODES_PER_W,), jnp.float32),
        pltpu.VMEM((NODES_PER_W,), jnp.float32),
        pltpu.VMEM((NODES_PER_W,), jnp.float32),
        pltpu.VMEM((NODES_PER_W * 16,), jnp.float32),
        pltpu.VMEM((NODES_PER_W * 16,), jnp.float32),
    ],
    compiler_params=_SC_PARAMS,
)(_prep_body)


PIECE = ROWS_PER_TILE // 8


def _prop_body(s_hbm, src_hbm, dst_hbm, aggA_hbm, aggB_hbm,
               sidx0, sidx1, sidx2, didx0, didx1, didx2, rows0, rows1,
               isem0, isem1, isem2, gsem0, gsem1, ssem0, ssem1, acc_sh):
    c = lax.axis_index("c")
    s = lax.axis_index("s")
    wid = c * 16 + s
    r0 = s * ROWS_PER_TILE
    sidx = (sidx0, sidx1, sidx2)
    didx = (didx0, didx1, didx2)
    rows = (rows0, rows1)
    isem = (isem0, isem1, isem2)
    gsem = (gsem0, gsem1)
    ssem = (ssem0, ssem1)

    def fillz(j, carry):
        rows0[j, :] = jnp.zeros((16,), jnp.float32)
        return carry

    lax.fori_loop(0, CHUNK, fillz, 0)
    for j in range(8):
        pltpu.sync_copy(rows0, acc_sh.at[pl.ds(r0 + j * PIECE, PIECE)])
    plsc.subcore_barrier()

    e0 = wid * EPW

    def idx_issue(base, t):
        off = base + t * CHUNK
        k = t % 3
        return (pltpu.async_copy(src_hbm.at[pl.ds(off, CHUNK)], sidx[k],
                                 isem[k]),
                pltpu.async_copy(dst_hbm.at[pl.ds(off, CHUNK)], didx[k],
                                 isem[k]))

    def group(g, carry):
        base = e0 + g * GROUP * CHUNK
        hidx = [None] * (GROUP + 1)
        hg = [None] * GROUP
        hs = [None] * GROUP
        hidx[0] = idx_issue(base, 0)
        for t in range(GROUP):
            b = t % 2
            if t >= 2:
                hs[t - 2].wait()
            for h in hidx[t]:
                h.wait()
            hg[t] = pltpu.async_copy(s_hbm.at[sidx[t % 3]], rows[b], gsem[b])
            if t + 1 < GROUP:
                hidx[t + 1] = idx_issue(base, t + 1)
            if t >= 1:
                hg[t - 1].wait()
                hs[t - 1] = pltpu.async_copy(
                    rows[(t - 1) % 2], acc_sh.at[didx[(t - 1) % 3]],
                    ssem[(t - 1) % 2], add=True)
        hg[GROUP - 1].wait()
        hs[GROUP - 1] = pltpu.async_copy(
            rows[(GROUP - 1) % 2], acc_sh.at[didx[(GROUP - 1) % 3]],
            ssem[(GROUP - 1) % 2], add=True)
        hs[GROUP - 2].wait()
        hs[GROUP - 1].wait()
        return carry

    lax.fori_loop(0, CHUNKS_PER_W // GROUP, group, 0)
    plsc.subcore_barrier()

    def dump(out_hbm):
        ha = [None] * 8
        hw = [None] * 8
        for j in range(8):
            b = j % 2
            if j >= 2:
                hw[j - 2].wait()
            ha[j] = pltpu.async_copy(acc_sh.at[pl.ds(r0 + j * PIECE, PIECE)],
                                     rows[b], gsem[b])
            if j >= 1:
                ha[j - 1].wait()
                hw[j - 1] = pltpu.async_copy(
                    rows[(j - 1) % 2],
                    out_hbm.at[pl.ds(r0 + (j - 1) * PIECE, PIECE)],
                    ssem[(j - 1) % 2])
        ha[7].wait()
        hw[7] = pltpu.async_copy(rows[1],
                                 out_hbm.at[pl.ds(r0 + 7 * PIECE, PIECE)],
                                 ssem[1])
        hw[6].wait()
        hw[7].wait()

    @pl.when(c == 0)
    def _():
        dump(aggA_hbm)

    @pl.when(c == 1)
    def _():
        dump(aggB_hbm)


_prop = functools.partial(
    pl.kernel,
    out_type=[
        jax.ShapeDtypeStruct((NPAD, F), jnp.float32),
        jax.ShapeDtypeStruct((NPAD, F), jnp.float32),
    ],
    mesh=_SC_MESH,
    scratch_types=[
        pltpu.VMEM((CHUNK,), jnp.int32),
        pltpu.VMEM((CHUNK,), jnp.int32),
        pltpu.VMEM((CHUNK,), jnp.int32),
        pltpu.VMEM((CHUNK,), jnp.int32),
        pltpu.VMEM((CHUNK,), jnp.int32),
        pltpu.VMEM((CHUNK,), jnp.int32),
        pltpu.VMEM((CHUNK, F), jnp.float32),
        pltpu.VMEM((CHUNK, F), jnp.float32),
        pltpu.SemaphoreType.DMA,
        pltpu.SemaphoreType.DMA,
        pltpu.SemaphoreType.DMA,
        pltpu.SemaphoreType.DMA,
        pltpu.SemaphoreType.DMA,
        pltpu.SemaphoreType.DMA,
        pltpu.SemaphoreType.DMA,
        pltpu.VMEM_SHARED((NPAD, F), jnp.float32),
    ],
    compiler_params=_SC_PARAMS,
)(_prop_body)


def _upd_body(aggA_ref, aggB_ref, s_ref, s0_ref, d2_ref, out_ref):
    g = aggA_ref[...] + aggB_ref[...] + s_ref[...]
    out_ref[...] = d2_ref[...] * g + ALPHA * s0_ref[...]


def _update(aggA2, aggB2, sf2, s0f2, d2f2):
    blk = L2 // 49
    spec = pl.BlockSpec((blk, 128), lambda i: (i, 0))
    return pl.pallas_call(
        _upd_body,
        grid=(49,),
        in_specs=[spec, spec, spec, spec, spec],
        out_specs=spec,
        out_shape=jax.ShapeDtypeStruct((L2, 128), jnp.float32),
    )(aggA2, aggB2, sf2, s0f2, d2f2)


def _fin_body(aggA_ref, aggB_ref, s_ref, s0_ref, dis_ref, out_ref):
    dis = dis_ref[...]
    g = aggA_ref[...] + aggB_ref[...] + s_ref[...]
    s_new = (1.0 - ALPHA) * dis * dis * g + ALPHA * s0_ref[...]
    o = s_new / dis
    parts = []
    for k in range(8):
        p = o[:, 16 * k:16 * k + 16]
        m = jnp.max(p, axis=1, keepdims=True)
        e = jnp.exp(p - m)
        lse = jnp.log(jnp.sum(e, axis=1, keepdims=True))
        parts.append(p - m - lse)
    out_ref[...] = jnp.concatenate(parts, axis=1)


def _final(aggA2, aggB2, sf2, s0f2, disf2):
    blk = 1024
    spec = pl.BlockSpec((blk, 128), lambda i: (i, 0))
    return pl.pallas_call(
        _fin_body,
        grid=(13,),
        in_specs=[spec, spec, spec, spec, spec],
        out_specs=spec,
        out_shape=jax.ShapeDtypeStruct((N_NODES // 8, 128), jnp.float32),
    )(aggA2, aggB2, sf2, s0f2, disf2)


def kernel(x, edge_index, W1, b1, W2, b2):
    src = edge_index[0].astype(jnp.int32)
    dst = edge_index[1].astype(jnp.int32)
    pad_idx = jnp.full((EPAD - N_EDGES,), N_NODES, jnp.int32)
    src_p = jnp.concatenate([src, pad_idx])
    dst_p = jnp.concatenate([dst, pad_idx])

    cnt = _hist(dst_p)
    disf, d2f = _prep(cnt)
    disf2 = disf.reshape(L2, 128)
    d2f2 = d2f.reshape(L2, 128)
    z2d = _mlp(x, W1, b1, W2, b2)
    s0f2 = _s0mul(disf2, z2d.reshape(-1).reshape(L2, 128))

    sf2 = s0f2
    for _ in range(K_PROP - 1):
        aggA, aggB = _prop(sf2.reshape(NPAD, F), src_p, dst_p)
        sf2 = _update(aggA.reshape(L2, 128), aggB.reshape(L2, 128),
                      sf2, s0f2, d2f2)
    aggA, aggB = _prop(sf2.reshape(NPAD, F), src_p, dst_p)
    outf = _final(aggA.reshape(L2, 128), aggB.reshape(L2, 128),
                  sf2, s0f2, disf2)
    return outf.reshape(N_NODES, F)


# speedup vs baseline: 96.5552x; 1.0725x over previous
import functools
import jax
import jax.numpy as jnp
from jax import lax
from jax.experimental import pallas as pl
from jax.experimental.pallas import tpu as pltpu
from jax.experimental.pallas import tpu_sc as plsc

K_PROP = 10
ALPHA = 0.1

N_NODES = 100000
N_EDGES = 3200000
F = 16
NPAD = 100352
NF = NPAD * F
ROWS_PER_TILE = NPAD // 16
NODES_PER_W = NPAD // 32

NW = 32
CHUNK = 784
CHUNKS_PER_W = 128
GROUP = 32
EPW = CHUNK * CHUNKS_PER_W
EPAD = EPW * NW

_SC_MESH = plsc.VectorSubcoreMesh(core_axis_name="c", subcore_axis_name="s")
_SC_PARAMS = pltpu.CompilerParams(use_tc_tiling_on_sc=False,
                                  needs_layout_passes=False)


L2 = NF // 128


def _mlp_body(x_ref, w1t_ref, b1_ref, w2t_ref, b2_ref, z_ref):
    h = jnp.dot(x_ref[...], w1t_ref[...], preferred_element_type=jnp.float32)
    h = jnp.maximum(h + b1_ref[...], 0.0)
    z = jnp.dot(h, w2t_ref[...], preferred_element_type=jnp.float32)
    z_ref[...] = z + b2_ref[...]


def _mlp(x, W1, b1, W2, b2):
    n, d = x.shape
    hidden = W1.shape[0]
    ncls = W2.shape[0]
    blk = 2000
    return pl.pallas_call(
        _mlp_body,
        grid=(n // blk,),
        in_specs=[
            pl.BlockSpec((blk, d), lambda i: (i, 0)),
            pl.BlockSpec((d, hidden), lambda i: (0, 0)),
            pl.BlockSpec((1, hidden), lambda i: (0, 0)),
            pl.BlockSpec((hidden, ncls), lambda i: (0, 0)),
            pl.BlockSpec((1, ncls), lambda i: (0, 0)),
        ],
        out_specs=pl.BlockSpec((blk, ncls), lambda i: (i, 0)),
        out_shape=jax.ShapeDtypeStruct((NPAD, ncls), jnp.float32),
    )(x, W1.T, b1[None, :], W2.T, b2[None, :])


def _s0_body(disf_ref, zf_ref, s0_ref):
    s0_ref[...] = disf_ref[...] * zf_ref[...]


def _s0mul(disf2, zf2):
    blk = L2 // 7
    spec = pl.BlockSpec((blk, 128), lambda i: (i, 0))
    return pl.pallas_call(
        _s0_body,
        grid=(7,),
        in_specs=[spec, spec],
        out_specs=spec,
        out_shape=jax.ShapeDtypeStruct((L2, 128), jnp.float32),
    )(disf2, zf2)


def _hist_body(dst_hbm, cnt_hbm, didx_v, ones_v, zeros_v, acc_sh):
    c = lax.axis_index("c")
    s = lax.axis_index("s")
    wid = c * 16 + s
    r0 = s * ROWS_PER_TILE

    def fill(i, carry):
        ones_v[pl.ds(i * 16, 16)] = jnp.full((16,), 1.0, jnp.float32)
        zeros_v[pl.ds(i * 16, 16)] = jnp.zeros((16,), jnp.float32)
        return carry

    lax.fori_loop(0, ROWS_PER_TILE // 16, fill, 0)
    pltpu.sync_copy(zeros_v, acc_sh.at[pl.ds(r0, ROWS_PER_TILE)])
    plsc.subcore_barrier()

    e0 = wid * EPW

    def chunk(i, carry):
        pltpu.sync_copy(dst_hbm.at[pl.ds(e0 + i * CHUNK, CHUNK)], didx_v)
        pltpu.sync_copy(ones_v.at[pl.ds(0, CHUNK)], acc_sh.at[didx_v], add=True)
        return carry

    lax.fori_loop(0, CHUNKS_PER_W, chunk, 0)
    plsc.subcore_barrier()
    pltpu.sync_copy(acc_sh.at[pl.ds(r0, ROWS_PER_TILE)], zeros_v)
    pltpu.sync_copy(zeros_v, cnt_hbm.at[pl.ds(c * NPAD + r0, ROWS_PER_TILE)])


_hist = functools.partial(
    pl.kernel,
    out_type=jax.ShapeDtypeStruct((2 * NPAD,), jnp.float32),
    mesh=_SC_MESH,
    scratch_types=[
        pltpu.VMEM((CHUNK,), jnp.int32),
        pltpu.VMEM((ROWS_PER_TILE,), jnp.float32),
        pltpu.VMEM((ROWS_PER_TILE,), jnp.float32),
        pltpu.VMEM_SHARED((NPAD,), jnp.float32),
    ],
    compiler_params=_SC_PARAMS,
)(_hist_body)


def _rsqrt16(x):
    i = plsc.bitcast(x, jnp.int32)
    i = jnp.int32(0x5F3759DF) - lax.shift_right_logical(i, 1)
    y = plsc.bitcast(i, jnp.float32)
    for _ in range(3):
        y = y * (1.5 - 0.5 * x * y * y)
    return y


def _prep_body(cnt_hbm, disf_hbm, d2f_hbm, ca_v, cb_v, dis_v, d2_v, ed_v, eq_v):
    c = lax.axis_index("c")
    s = lax.axis_index("s")
    wid = c * 16 + s
    n0 = wid * NODES_PER_W
    pltpu.sync_copy(cnt_hbm.at[pl.ds(n0, NODES_PER_W)], ca_v)
    pltpu.sync_copy(cnt_hbm.at[pl.ds(NPAD + n0, NODES_PER_W)], cb_v)

    def compute(i, carry):
        o = pl.ds(i * 16, 16)
        deg = ca_v[o] + cb_v[o] + 1.0
        dis = _rsqrt16(deg)
        dis_v[o] = dis
        d2_v[o] = (1.0 - ALPHA) * dis * dis
        return carry

    lax.fori_loop(0, NODES_PER_W // 16, compute, 0)

    def expand(j, carry):
        idx = jnp.full((16,), j, jnp.int32)
        ed_v[pl.ds(j * 16, 16)] = plsc.load_gather(dis_v, [idx])
        eq_v[pl.ds(j * 16, 16)] = plsc.load_gather(d2_v, [idx])
        return carry

    lax.fori_loop(0, NODES_PER_W, expand, 0)
    pltpu.sync_copy(ed_v, disf_hbm.at[pl.ds(n0 * 16, NODES_PER_W * 16)])
    pltpu.sync_copy(eq_v, d2f_hbm.at[pl.ds(n0 * 16, NODES_PER_W * 16)])


_prep = functools.partial(
    pl.kernel,
    out_type=[
        jax.ShapeDtypeStruct((NF,), jnp.float32),
        jax.ShapeDtypeStruct((NF,), jnp.float32),
    ],
    mesh=_SC_MESH,
    scratch_types=[
        pltpu.VMEM((NODES_PER_W,), jnp.float32),
        pltpu.VMEM((NODES_PER_W,), jnp.float32),
        pltpu.VMEM((NODES_PER_W,), jnp.float32),
        pltpu.VMEM((NODES_PER_W,), jnp.float32),
        pltpu.VMEM((NODES_PER_W * 16,), jnp.float32),
        pltpu.VMEM((NODES_PER_W * 16,), jnp.float32),
    ],
    compiler_params=_SC_PARAMS,
)(_prep_body)


PIECE = ROWS_PER_TILE // 8


def _prop_body(s_hbm, src_hbm, dst_hbm, aggA_hbm, aggB_hbm,
               sidx0, sidx1, sidx2, didx0, didx1, didx2, rows0, rows1,
               isem0, isem1, isem2, gsem0, gsem1, ssem0, ssem1, acc_sh):
    c = lax.axis_index("c")
    s = lax.axis_index("s")
    wid = c * 16 + s
    r0 = s * ROWS_PER_TILE
    sidx = (sidx0, sidx1, sidx2)
    didx = (didx0, didx1, didx2)
    rows = (rows0, rows1)
    isem = (isem0, isem1, isem2)
    gsem = (gsem0, gsem1)
    ssem = (ssem0, ssem1)

    def fillz(j, carry):
        rows0[j, :] = jnp.zeros((16,), jnp.float32)
        return carry

    lax.fori_loop(0, CHUNK, fillz, 0)
    for j in range(8):
        pltpu.sync_copy(rows0, acc_sh.at[pl.ds(r0 + j * PIECE, PIECE)])
    plsc.subcore_barrier()

    e0 = wid * EPW

    def idx_issue(base, t):
        off = base + t * CHUNK
        k = t % 3
        return (pltpu.async_copy(src_hbm.at[pl.ds(off, CHUNK)], sidx[k],
                                 isem[k]),
                pltpu.async_copy(dst_hbm.at[pl.ds(off, CHUNK)], didx[k],
                                 isem[k]))

    def group(g, carry):
        base = e0 + g * GROUP * CHUNK
        hidx = [None] * (GROUP + 1)
        hg = [None] * GROUP
        hs = [None] * GROUP
        hidx[0] = idx_issue(base, 0)
        for t in range(GROUP):
            b = t % 2
            if t >= 2:
                hs[t - 2].wait()
            for h in hidx[t]:
                h.wait()
            hg[t] = pltpu.async_copy(s_hbm.at[sidx[t % 3]], rows[b], gsem[b])
            if t + 1 < GROUP:
                hidx[t + 1] = idx_issue(base, t + 1)
            if t >= 1:
                hg[t - 1].wait()
                hs[t - 1] = pltpu.async_copy(
                    rows[(t - 1) % 2], acc_sh.at[didx[(t - 1) % 3]],
                    ssem[(t - 1) % 2], add=True)
        hg[GROUP - 1].wait()
        hs[GROUP - 1] = pltpu.async_copy(
            rows[(GROUP - 1) % 2], acc_sh.at[didx[(GROUP - 1) % 3]],
            ssem[(GROUP - 1) % 2], add=True)
        hs[GROUP - 2].wait()
        hs[GROUP - 1].wait()
        return carry

    lax.fori_loop(0, CHUNKS_PER_W // GROUP, group, 0)
    plsc.subcore_barrier()

    def dump(out_hbm):
        ha = [None] * 8
        hw = [None] * 8
        for j in range(8):
            b = j % 2
            if j >= 2:
                hw[j - 2].wait()
            ha[j] = pltpu.async_copy(acc_sh.at[pl.ds(r0 + j * PIECE, PIECE)],
                                     rows[b], gsem[b])
            if j >= 1:
                ha[j - 1].wait()
                hw[j - 1] = pltpu.async_copy(
                    rows[(j - 1) % 2],
                    out_hbm.at[pl.ds(r0 + (j - 1) * PIECE, PIECE)],
                    ssem[(j - 1) % 2])
        ha[7].wait()
        hw[7] = pltpu.async_copy(rows[1],
                                 out_hbm.at[pl.ds(r0 + 7 * PIECE, PIECE)],
                                 ssem[1])
        hw[6].wait()
        hw[7].wait()

    @pl.when(c == 0)
    def _():
        dump(aggA_hbm)

    @pl.when(c == 1)
    def _():
        dump(aggB_hbm)


_prop = functools.partial(
    pl.kernel,
    out_type=[
        jax.ShapeDtypeStruct((NPAD, F), jnp.float32),
        jax.ShapeDtypeStruct((NPAD, F), jnp.float32),
    ],
    mesh=_SC_MESH,
    scratch_types=[
        pltpu.VMEM((CHUNK,), jnp.int32),
        pltpu.VMEM((CHUNK,), jnp.int32),
        pltpu.VMEM((CHUNK,), jnp.int32),
        pltpu.VMEM((CHUNK,), jnp.int32),
        pltpu.VMEM((CHUNK,), jnp.int32),
        pltpu.VMEM((CHUNK,), jnp.int32),
        pltpu.VMEM((CHUNK, F), jnp.float32),
        pltpu.VMEM((CHUNK, F), jnp.float32),
        pltpu.SemaphoreType.DMA,
        pltpu.SemaphoreType.DMA,
        pltpu.SemaphoreType.DMA,
        pltpu.SemaphoreType.DMA,
        pltpu.SemaphoreType.DMA,
        pltpu.SemaphoreType.DMA,
        pltpu.SemaphoreType.DMA,
        pltpu.VMEM_SHARED((NPAD, F), jnp.float32),
    ],
    compiler_params=_SC_PARAMS,
)(_prop_body)


def _upd_body(aggA_ref, aggB_ref, s_ref, s0_ref, d2_ref, out_ref):
    g = aggA_ref[...] + aggB_ref[...] + s_ref[...]
    out_ref[...] = d2_ref[...] * g + ALPHA * s0_ref[...]


def _update(aggA2, aggB2, sf2, s0f2, d2f2):
    blk = L2 // 7
    spec = pl.BlockSpec((blk, 128), lambda i: (i, 0))
    return pl.pallas_call(
        _upd_body,
        grid=(7,),
        in_specs=[spec, spec, spec, spec, spec],
        out_specs=spec,
        out_shape=jax.ShapeDtypeStruct((L2, 128), jnp.float32),
    )(aggA2, aggB2, sf2, s0f2, d2f2)


def _fin_body(aggA_ref, aggB_ref, s_ref, s0_ref, dis_ref, out_ref):
    dis = dis_ref[...]
    g = aggA_ref[...] + aggB_ref[...] + s_ref[...]
    s_new = (1.0 - ALPHA) * dis * dis * g + ALPHA * s0_ref[...]
    o = s_new / dis
    parts = []
    for k in range(8):
        p = o[:, 16 * k:16 * k + 16]
        m = jnp.max(p, axis=1, keepdims=True)
        e = jnp.exp(p - m)
        lse = jnp.log(jnp.sum(e, axis=1, keepdims=True))
        parts.append(p - m - lse)
    out_ref[...] = jnp.concatenate(parts, axis=1)


def _final(aggA2, aggB2, sf2, s0f2, disf2):
    blk = 1024
    spec = pl.BlockSpec((blk, 128), lambda i: (i, 0))
    return pl.pallas_call(
        _fin_body,
        grid=(13,),
        in_specs=[spec, spec, spec, spec, spec],
        out_specs=spec,
        out_shape=jax.ShapeDtypeStruct((N_NODES // 8, 128), jnp.float32),
    )(aggA2, aggB2, sf2, s0f2, disf2)


def kernel(x, edge_index, W1, b1, W2, b2):
    src = edge_index[0].astype(jnp.int32)
    dst = edge_index[1].astype(jnp.int32)
    pad_idx = jnp.full((EPAD - N_EDGES,), N_NODES, jnp.int32)
    src_p = jnp.concatenate([src, pad_idx])
    dst_p = jnp.concatenate([dst, pad_idx])

    cnt = _hist(dst_p)
    disf, d2f = _prep(cnt)
    disf2 = disf.reshape(L2, 128)
    d2f2 = d2f.reshape(L2, 128)
    z2d = _mlp(x, W1, b1, W2, b2)
    s0f2 = _s0mul(disf2, z2d.reshape(-1).reshape(L2, 128))

    sf2 = s0f2
    for _ in range(K_PROP - 1):
        aggA, aggB = _prop(sf2.reshape(NPAD, F), src_p, dst_p)
        sf2 = _update(aggA.reshape(L2, 128), aggB.reshape(L2, 128),
                      sf2, s0f2, d2f2)
    aggA, aggB = _prop(sf2.reshape(NPAD, F), src_p, dst_p)
    outf = _final(aggA.reshape(L2, 128), aggB.reshape(L2, 128),
                  sf2, s0f2, disf2)
    return outf.reshape(N_NODES, F)
